# baseline (speedup 1.0000x reference)
import functools

import jax
import jax.numpy as jnp
from jax import lax
from jax.experimental import pallas as pl
from jax.experimental.pallas import tpu as pltpu
from jax.experimental.pallas import tpu_sc as plsc

_NC = 2
_NS = 16
_NW = _NC * _NS
_L = 16
_E = 1024
_C = 32


@functools.partial(jax.jit, static_argnums=(3,))
def _run(seqs2d, idx3d, weight, total_rows):
    rows_per_worker = total_rows // _NW
    nch = rows_per_worker // _C
    mesh = plsc.VectorSubcoreMesh(
        core_axis_name="c", subcore_axis_name="s", num_cores=_NC, num_subcores=_NS
    )

    @functools.partial(
        pl.kernel,
        out_type=jax.ShapeDtypeStruct((total_rows, _E), jnp.float32),
        mesh=mesh,
        scratch_types=[
            pltpu.VMEM((nch, _C), jnp.int32),
            pltpu.VMEM((_C, _E), jnp.float32),
            pltpu.VMEM((_C, _E), jnp.float32),
            pltpu.SemaphoreType.DMA,
        ],
    )
    def k(seqs_hbm, idx_hbm, w_hbm, out_hbm, idx_v, sbuf, wbuf, sem_w):
        wid = lax.axis_index("s") * _NC + lax.axis_index("c")
        base = wid * rows_per_worker

        pltpu.sync_copy(idx_hbm.at[wid], idx_v)

        def bump(j, carry):
            for t in range(_C // _L):
                idx_v[j, pl.ds(t * _L, _L)] = idx_v[j, pl.ds(t * _L, _L)] + 1
            return carry

        lax.fori_loop(0, nch, bump, 0)

        def chunk(j, carry):
            row0 = base + j * _C
            pltpu.sync_copy(seqs_hbm.at[pl.ds(row0, _C)], sbuf)
            pltpu.async_copy(w_hbm.at[idx_v.at[j]], wbuf, sem_w).wait()

            def add_row(r, c2):
                for t in range(_E // _L):
                    sl = pl.ds(t * _L, _L)
                    sbuf[r, sl] = sbuf[r, sl] + wbuf[r, sl]
                return c2

            lax.fori_loop(0, _C, add_row, 0)
            pltpu.sync_copy(sbuf, out_hbm.at[pl.ds(row0, _C)])
            return carry

        lax.fori_loop(0, nch, chunk, 0)

    return k(seqs2d, idx3d, weight)


def kernel(seqs, position_indices, weight):
    b, s, e = seqs.shape
    total_rows = b * s
    rows_per_worker = total_rows // _NW
    seqs2d = seqs.reshape(total_rows, e)
    idx3d = position_indices.reshape(_NW, rows_per_worker // _C, _C).astype(
        jnp.int32
    )
    out = _run(seqs2d, idx3d, weight, total_rows)
    return out.reshape(b, s, e)


# speedup vs baseline: 1.9277x; 1.9277x over previous
import functools

import jax
import jax.numpy as jnp
from jax import lax
from jax.experimental import pallas as pl
from jax.experimental.pallas import tpu as pltpu
from jax.experimental.pallas import tpu_sc as plsc

_NC = 2
_NS = 16
_NW = _NC * _NS
_L = 16
_E = 1024
_C = 8
_NBUF = 4


@functools.partial(jax.jit, static_argnums=(3,))
def _run(seqs2d, idx2d, weight, total_rows):
    rows_per_worker = total_rows // _NW
    nch = rows_per_worker // _C
    mesh = plsc.VectorSubcoreMesh(
        core_axis_name="c", subcore_axis_name="s", num_cores=_NC, num_subcores=_NS
    )

    @functools.partial(
        pl.kernel,
        out_type=jax.ShapeDtypeStruct((total_rows, _E), jnp.float32),
        mesh=mesh,
        scratch_types=[
            pltpu.VMEM((rows_per_worker,), jnp.int32),
            pltpu.VMEM((_NBUF, _C, _E), jnp.float32),
            pltpu.VMEM((_NBUF, _C, _E), jnp.float32),
            [pltpu.SemaphoreType.DMA] * _NBUF,
            [pltpu.SemaphoreType.DMA] * _NBUF,
        ],
    )
    def k(seqs_hbm, idx_hbm, w_hbm, out_hbm, idx_v, sbuf, wbuf, sis, sos):
        wid = lax.axis_index("s") * _NC + lax.axis_index("c")
        base = wid * rows_per_worker

        pltpu.sync_copy(idx_hbm.at[wid], idx_v)

        def bump(i, carry):
            sl = pl.ds(pl.multiple_of(i * _L, _L), _L)
            idx_v[sl] = idx_v[sl] + 1
            return carry

        lax.fori_loop(0, rows_per_worker // _L, bump, 0)

        def issue_in(j, slot):
            row0 = base + j * _C
            off = pl.multiple_of(j * _C, _C)
            pltpu.async_copy(
                seqs_hbm.at[pl.ds(row0, _C)], sbuf.at[slot], sis[slot]
            )
            pltpu.async_copy(
                w_hbm.at[idx_v.at[pl.ds(off, _C)]], wbuf.at[slot], sis[slot]
            )

        def wait_in(j, slot):
            row0 = base + j * _C
            pltpu.make_async_copy(
                seqs_hbm.at[pl.ds(row0, _C)], sbuf.at[slot], sis[slot]
            ).wait()
            pltpu.make_async_copy(
                w_hbm.at[idx_v.at[pl.ds(0, _C)]], wbuf.at[slot], sis[slot]
            ).wait()

        def issue_out(j, slot):
            row0 = base + j * _C
            pltpu.async_copy(
                sbuf.at[slot], out_hbm.at[pl.ds(row0, _C)], sos[slot]
            )

        def wait_out(j, slot):
            row0 = base + j * _C
            pltpu.make_async_copy(
                sbuf.at[slot], out_hbm.at[pl.ds(row0, _C)], sos[slot]
            ).wait()

        issue_in(0, 0)
        issue_in(1, 1)

        def super_step(jo, carry):
            for b in range(_NBUF):
                j = jo * _NBUF + b
                bn = (b + 2) % _NBUF

                @pl.when(j + 2 < nch)
                def _():
                    @pl.when(j + 2 >= _NBUF)
                    def _():
                        wait_out(j - 2, bn)

                    issue_in(j + 2, bn)

                wait_in(j, b)

                def add_row(r, c2):
                    for t in range(_E // _L):
                        sl = pl.ds(t * _L, _L)
                        plsc.addupdate(sbuf.at[b, r, sl], wbuf[b, r, sl])
                    return c2

                lax.fori_loop(0, _C, add_row, 0)
                issue_out(j, b)
            return carry

        lax.fori_loop(0, nch // _NBUF, super_step, 0)

        wait_out(nch - 2, (nch - 2) % _NBUF)
        wait_out(nch - 1, (nch - 1) % _NBUF)

    return k(seqs2d, idx2d, weight)


def kernel(seqs, position_indices, weight):
    b, s, e = seqs.shape
    total_rows = b * s
    seqs2d = seqs.reshape(total_rows, e)
    idx2d = position_indices.reshape(_NW, total_rows // _NW).astype(jnp.int32)
    out = _run(seqs2d, idx2d, weight, total_rows)
    return out.reshape(b, s, e)
